# Initial kernel scaffold; baseline (speedup 1.0000x reference)
import jax
import jax.numpy as jnp
from jax.experimental import pallas as pl


def kernel(ctx, expert_idx, dec_W, dec_b, copy_W1, copy_b1, copy_W2, copy_b2, psr_lut, atk_lut, ori_psr, ori_atk):
    raise NotImplementedError("write your pallas kernel here")


# speedup vs baseline: 2.9247x; 2.9247x over previous
import functools

import jax
import jax.numpy as jnp
from jax import lax
from jax.experimental import pallas as pl
from jax.experimental.pallas import tpu as pltpu
from jax.experimental.pallas import tpu_sc as plsc

TOK = 4096
HS = 1024
E = 8
M = 512
D = 256
TPE = TOK // E

NC = 2
NS = 16
NW = NC * NS
RPW = TOK // NW
CH = 4
CR = RPW // CH

@functools.lru_cache(maxsize=None)
def _make_sc_gather():
    mesh = plsc.VectorSubcoreMesh(core_axis_name="c", subcore_axis_name="s")

    @functools.partial(
        pl.kernel,
        out_type=[
            jax.ShapeDtypeStruct((TOK, HS), jnp.float32),
            jax.ShapeDtypeStruct((TOK, D), jnp.float32),
            jax.ShapeDtypeStruct((TOK, D), jnp.float32),
        ],
        mesh=mesh,
        scratch_types=[
            pltpu.VMEM((CH, CR), jnp.int32),
            pltpu.VMEM((RPW,), jnp.int32),
            pltpu.VMEM((CR, HS), jnp.float32),
            pltpu.VMEM((RPW, D), jnp.float32),
            pltpu.SemaphoreType.DMA,
        ],
    )
    def _sc_gather(ctx_hbm, psr_hbm, atk_hbm, p3_hbm, p2_hbm,
                   xs_hbm, ps_hbm, as_hbm,
                   idxc_v, idx_v, rows_v, emb_v, sem):
        wid = lax.axis_index("s") * NC + lax.axis_index("c")
        base = wid * RPW
        pltpu.sync_copy(p3_hbm.at[wid], idxc_v)
        for c in range(CH):
            pltpu.async_copy(ctx_hbm.at[idxc_v.at[c]], rows_v, sem).wait()
            pltpu.sync_copy(rows_v, xs_hbm.at[pl.ds(base + c * CR, CR)])
        pltpu.sync_copy(p2_hbm.at[wid], idx_v)
        pltpu.async_copy(psr_hbm.at[idx_v], emb_v, sem).wait()
        pltpu.sync_copy(emb_v, ps_hbm.at[pl.ds(base, RPW)])
        pltpu.async_copy(atk_hbm.at[idx_v], emb_v, sem).wait()
        pltpu.sync_copy(emb_v, as_hbm.at[pl.ds(base, RPW)])

    return _sc_gather


@functools.lru_cache(maxsize=None)
def _make_sc_scatter():
    mesh = plsc.VectorSubcoreMesh(core_axis_name="c", subcore_axis_name="s")

    @functools.partial(
        pl.kernel,
        out_type=jax.ShapeDtypeStruct((TOK, 2 * D), jnp.float32),
        mesh=mesh,
        scratch_types=[
            pltpu.VMEM((RPW,), jnp.int32),
            pltpu.VMEM((RPW, 2 * D), jnp.float32),
            pltpu.SemaphoreType.DMA,
        ],
    )
    def _sc_scatter(blend_hbm, p2_hbm, out_hbm, idx_v, rows_v, sem):
        wid = lax.axis_index("s") * NC + lax.axis_index("c")
        base = wid * RPW
        pltpu.sync_copy(p2_hbm.at[wid], idx_v)
        pltpu.sync_copy(blend_hbm.at[pl.ds(base, RPW)], rows_v)
        pltpu.async_copy(rows_v, out_hbm.at[idx_v], sem).wait()

    return _sc_scatter


def _expert_body(x_ref, opsr_ref, oatk_ref, dW_ref, db_ref, plut_ref,
                 alut_ref, w1_ref, b1_ref, w2_ref, b2_ref, out_ref, ent_ref):
    x = x_ref[...]
    logits = jnp.dot(x, dW_ref[0], preferred_element_type=jnp.float32) + db_ref[0]
    m = jnp.max(logits, axis=-1, keepdims=True)
    z = logits - m
    ez = jnp.exp(z)
    s = jnp.sum(ez, axis=-1, keepdims=True)
    spt = ez / s
    pspt = z - jnp.log(s)
    ent_blk = jnp.sum(-pspt * spt) * (1.0 / (TPE * M))

    @pl.when(pl.program_id(0) == 0)
    def _():
        ent_ref[0, 0] = 0.0

    ent_ref[0, 0] += ent_blk

    psr = jnp.dot(spt, plut_ref[0], preferred_element_type=jnp.float32)
    atk = jnp.dot(spt, alut_ref[0], preferred_element_type=jnp.float32)

    h = jnp.maximum(jnp.dot(x, w1_ref[...], preferred_element_type=jnp.float32)
                    + b1_ref[...], 0.0)
    u = jnp.dot(h, w2_ref[...], preferred_element_type=jnp.float32) + b2_ref[...]
    um = jnp.max(u, axis=-1, keepdims=True)
    ue = jnp.exp(u - um)
    c = ue / jnp.sum(ue, axis=-1, keepdims=True)
    c0 = c[:, 0:1]
    c1 = c[:, 1:2]
    out_ref[:, :D] = opsr_ref[...] * c0 + psr * c1
    out_ref[:, D:] = oatk_ref[...] * c0 + atk * c1


def _tc_experts(xs, ops, oat, dec_W, dec_b, psr_lut, atk_lut, w1, b1, w2, b2):
    return pl.pallas_call(
        _expert_body,
        grid=(E,),
        in_specs=[
            pl.BlockSpec((TPE, HS), lambda e: (e, 0)),
            pl.BlockSpec((TPE, D), lambda e: (e, 0)),
            pl.BlockSpec((TPE, D), lambda e: (e, 0)),
            pl.BlockSpec((1, HS, M), lambda e: (e, 0, 0)),
            pl.BlockSpec((1, 1, M), lambda e: (e, 0, 0)),
            pl.BlockSpec((1, M, D), lambda e: (e, 0, 0)),
            pl.BlockSpec((1, M, D), lambda e: (e, 0, 0)),
            pl.BlockSpec((HS, 64), lambda e: (0, 0)),
            pl.BlockSpec((1, 64), lambda e: (0, 0)),
            pl.BlockSpec((64, 2), lambda e: (0, 0)),
            pl.BlockSpec((1, 2), lambda e: (0, 0)),
        ],
        out_specs=[
            pl.BlockSpec((TPE, 2 * D), lambda e: (e, 0)),
            pl.BlockSpec((1, 1), lambda e: (0, 0), memory_space=pltpu.SMEM),
        ],
        out_shape=[
            jax.ShapeDtypeStruct((TOK, 2 * D), jnp.float32),
            jax.ShapeDtypeStruct((1, 1), jnp.float32),
        ],
    )(xs, ops, oat, dec_W, dec_b, psr_lut, atk_lut, w1, b1, w2, b2)


def _add_ent_body(y_ref, ent_ref, o_ref):
    o_ref[...] = y_ref[...] + ent_ref[0, 0]


def _tc_add_ent(y, ent):
    return pl.pallas_call(
        _add_ent_body,
        grid=(E,),
        in_specs=[
            pl.BlockSpec((TPE, 2 * D), lambda i: (i, 0)),
            pl.BlockSpec((1, 1), lambda i: (0, 0), memory_space=pltpu.SMEM),
        ],
        out_specs=pl.BlockSpec((TPE, 2 * D), lambda i: (i, 0)),
        out_shape=jax.ShapeDtypeStruct((TOK, 2 * D), jnp.float32),
    )(y, ent)


def kernel(ctx, expert_idx, dec_W, dec_b, copy_W1, copy_b1, copy_W2, copy_b2,
           psr_lut, atk_lut, ori_psr, ori_atk):
    p = expert_idx.reshape(TOK).astype(jnp.int32)
    p3 = p.reshape(NW, CH, CR)
    p2 = p.reshape(NW, RPW)
    xs, ops, oat = _make_sc_gather()(ctx, ori_psr, ori_atk, p3, p2)
    blended, ent = _tc_experts(
        xs, ops, oat, dec_W, dec_b.reshape(E, 1, M), psr_lut, atk_lut,
        copy_W1, copy_b1.reshape(1, 64), copy_W2, copy_b2.reshape(1, 2))
    scattered = _make_sc_scatter()(blended, p2)
    return _tc_add_ent(scattered, ent)


# speedup vs baseline: 2.9347x; 1.0034x over previous
import functools

import jax
import jax.numpy as jnp
from jax import lax
from jax.experimental import pallas as pl
from jax.experimental.pallas import tpu as pltpu
from jax.experimental.pallas import tpu_sc as plsc

TOK = 4096
HS = 1024
E = 8
M = 512
D = 256
TPE = TOK // E

NC = 2
NS = 16
NW = NC * NS
RPW = TOK // NW
CH = 4
CR = RPW // CH

@functools.lru_cache(maxsize=None)
def _make_sc_gather():
    mesh = plsc.VectorSubcoreMesh(core_axis_name="c", subcore_axis_name="s")

    @functools.partial(
        pl.kernel,
        out_type=[
            jax.ShapeDtypeStruct((TOK, HS), jnp.float32),
            jax.ShapeDtypeStruct((TOK, D), jnp.float32),
            jax.ShapeDtypeStruct((TOK, D), jnp.float32),
        ],
        mesh=mesh,
        scratch_types=[
            pltpu.VMEM((CH, CR), jnp.int32),
            pltpu.VMEM((RPW,), jnp.int32),
            pltpu.VMEM((CR, HS), jnp.float32),
            pltpu.VMEM((RPW, D), jnp.float32),
            pltpu.SemaphoreType.DMA,
        ],
    )
    def _sc_gather(ctx_hbm, psr_hbm, atk_hbm, p3_hbm, p2_hbm,
                   xs_hbm, ps_hbm, as_hbm,
                   idxc_v, idx_v, rows_v, emb_v, sem):
        wid = lax.axis_index("s") * NC + lax.axis_index("c")
        base = wid * RPW
        pltpu.sync_copy(p3_hbm.at[wid], idxc_v)
        for c in range(CH):
            pltpu.async_copy(ctx_hbm.at[idxc_v.at[c]], rows_v, sem).wait()
            pltpu.sync_copy(rows_v, xs_hbm.at[pl.ds(base + c * CR, CR)])
        pltpu.sync_copy(p2_hbm.at[wid], idx_v)
        pltpu.async_copy(psr_hbm.at[idx_v], emb_v, sem).wait()
        pltpu.sync_copy(emb_v, ps_hbm.at[pl.ds(base, RPW)])
        pltpu.async_copy(atk_hbm.at[idx_v], emb_v, sem).wait()
        pltpu.sync_copy(emb_v, as_hbm.at[pl.ds(base, RPW)])

    return _sc_gather


@functools.lru_cache(maxsize=None)
def _make_sc_scatter():
    mesh = plsc.VectorSubcoreMesh(core_axis_name="c", subcore_axis_name="s")

    @functools.partial(
        pl.kernel,
        out_type=jax.ShapeDtypeStruct((TOK, 2 * D), jnp.float32),
        mesh=mesh,
        scratch_types=[
            pltpu.VMEM((RPW,), jnp.int32),
            pltpu.VMEM((RPW, 2 * D), jnp.float32),
            pltpu.SemaphoreType.DMA,
        ],
    )
    def _sc_scatter(blend_hbm, p2_hbm, out_hbm, idx_v, rows_v, sem):
        wid = lax.axis_index("s") * NC + lax.axis_index("c")
        base = wid * RPW
        pltpu.sync_copy(p2_hbm.at[wid], idx_v)
        pltpu.sync_copy(blend_hbm.at[pl.ds(base, RPW)], rows_v)
        pltpu.async_copy(rows_v, out_hbm.at[idx_v], sem).wait()

    return _sc_scatter


def _expert_body(x_ref, opsr_ref, oatk_ref, dW_ref, db_ref, plut_ref,
                 alut_ref, w1_ref, b1_ref, w2_ref, b2_ref, out_ref, ent_ref):
    x = x_ref[...]
    xb = x.astype(jnp.bfloat16)
    logits = jnp.dot(xb, dW_ref[0].astype(jnp.bfloat16),
                     preferred_element_type=jnp.float32) + db_ref[0]
    m = jnp.max(logits, axis=-1, keepdims=True)
    z = logits - m
    ez = jnp.exp(z)
    s = jnp.sum(ez, axis=-1, keepdims=True)
    spt = ez / s
    pspt = z - jnp.log(s)
    ent_blk = jnp.sum(-pspt * spt) * (1.0 / (TPE * M))

    @pl.when(pl.program_id(0) == 0)
    def _():
        ent_ref[0, 0] = 0.0

    ent_ref[0, 0] += ent_blk

    sptb = spt.astype(jnp.bfloat16)
    psr = jnp.dot(sptb, plut_ref[0].astype(jnp.bfloat16),
                  preferred_element_type=jnp.float32)
    atk = jnp.dot(sptb, alut_ref[0].astype(jnp.bfloat16),
                  preferred_element_type=jnp.float32)

    h = jnp.maximum(jnp.dot(xb, w1_ref[...].astype(jnp.bfloat16),
                            preferred_element_type=jnp.float32)
                    + b1_ref[...], 0.0)
    u = jnp.dot(h.astype(jnp.bfloat16), w2_ref[...].astype(jnp.bfloat16),
                preferred_element_type=jnp.float32) + b2_ref[...]
    um = jnp.max(u, axis=-1, keepdims=True)
    ue = jnp.exp(u - um)
    c = ue / jnp.sum(ue, axis=-1, keepdims=True)
    c0 = c[:, 0:1]
    c1 = c[:, 1:2]
    out_ref[:, :D] = opsr_ref[...] * c0 + psr * c1
    out_ref[:, D:] = oatk_ref[...] * c0 + atk * c1


def _tc_experts(xs, ops, oat, dec_W, dec_b, psr_lut, atk_lut, w1, b1, w2, b2):
    return pl.pallas_call(
        _expert_body,
        grid=(E,),
        in_specs=[
            pl.BlockSpec((TPE, HS), lambda e: (e, 0)),
            pl.BlockSpec((TPE, D), lambda e: (e, 0)),
            pl.BlockSpec((TPE, D), lambda e: (e, 0)),
            pl.BlockSpec((1, HS, M), lambda e: (e, 0, 0)),
            pl.BlockSpec((1, 1, M), lambda e: (e, 0, 0)),
            pl.BlockSpec((1, M, D), lambda e: (e, 0, 0)),
            pl.BlockSpec((1, M, D), lambda e: (e, 0, 0)),
            pl.BlockSpec((HS, 64), lambda e: (0, 0)),
            pl.BlockSpec((1, 64), lambda e: (0, 0)),
            pl.BlockSpec((64, 2), lambda e: (0, 0)),
            pl.BlockSpec((1, 2), lambda e: (0, 0)),
        ],
        out_specs=[
            pl.BlockSpec((TPE, 2 * D), lambda e: (e, 0)),
            pl.BlockSpec((1, 1), lambda e: (0, 0), memory_space=pltpu.SMEM),
        ],
        out_shape=[
            jax.ShapeDtypeStruct((TOK, 2 * D), jnp.float32),
            jax.ShapeDtypeStruct((1, 1), jnp.float32),
        ],
    )(xs, ops, oat, dec_W, dec_b, psr_lut, atk_lut, w1, b1, w2, b2)


def _add_ent_body(y_ref, ent_ref, o_ref):
    o_ref[...] = y_ref[...] + ent_ref[0, 0]


def _tc_add_ent(y, ent):
    return pl.pallas_call(
        _add_ent_body,
        grid=(E,),
        in_specs=[
            pl.BlockSpec((TPE, 2 * D), lambda i: (i, 0)),
            pl.BlockSpec((1, 1), lambda i: (0, 0), memory_space=pltpu.SMEM),
        ],
        out_specs=pl.BlockSpec((TPE, 2 * D), lambda i: (i, 0)),
        out_shape=jax.ShapeDtypeStruct((TOK, 2 * D), jnp.float32),
    )(y, ent)


def kernel(ctx, expert_idx, dec_W, dec_b, copy_W1, copy_b1, copy_W2, copy_b2,
           psr_lut, atk_lut, ori_psr, ori_atk):
    p = expert_idx.reshape(TOK).astype(jnp.int32)
    p3 = p.reshape(NW, CH, CR)
    p2 = p.reshape(NW, RPW)
    xs, ops, oat = _make_sc_gather()(ctx, ori_psr, ori_atk, p3, p2)
    blended, ent = _tc_experts(
        xs, ops, oat, dec_W, dec_b.reshape(E, 1, M), psr_lut, atk_lut,
        copy_W1, copy_b1.reshape(1, 64), copy_W2, copy_b2.reshape(1, 2))
    scattered = _make_sc_scatter()(blended, p2)
    return _tc_add_ent(scattered, ent)


# speedup vs baseline: 3.2188x; 1.0968x over previous
import functools

import jax
import jax.numpy as jnp
from jax import lax
from jax.experimental import pallas as pl
from jax.experimental.pallas import tpu as pltpu
from jax.experimental.pallas import tpu_sc as plsc

TOK = 4096
HS = 1024
E = 8
M = 512
D = 256
TPE = TOK // E

NC = 2
NS = 16
NW = NC * NS
RPW = TOK // NW
CH = 4
CR = RPW // CH

@functools.lru_cache(maxsize=None)
def _make_sc_gather():
    mesh = plsc.VectorSubcoreMesh(core_axis_name="c", subcore_axis_name="s")

    @functools.partial(
        pl.kernel,
        out_type=[
            jax.ShapeDtypeStruct((TOK, HS), jnp.float32),
            jax.ShapeDtypeStruct((TOK, D), jnp.float32),
            jax.ShapeDtypeStruct((TOK, D), jnp.float32),
        ],
        mesh=mesh,
        scratch_types=[
            pltpu.VMEM((CH, CR), jnp.int32),
            pltpu.VMEM((RPW,), jnp.int32),
            pltpu.VMEM((CR, HS), jnp.float32),
            pltpu.VMEM((RPW, D), jnp.float32),
            pltpu.SemaphoreType.DMA,
        ],
    )
    def _sc_gather(ctx_hbm, psr_hbm, atk_hbm, p3_hbm, p2_hbm,
                   xs_hbm, ps_hbm, as_hbm,
                   idxc_v, idx_v, rows_v, emb_v, sem):
        wid = lax.axis_index("s") * NC + lax.axis_index("c")
        base = wid * RPW
        pltpu.sync_copy(p3_hbm.at[wid], idxc_v)
        for c in range(CH):
            pltpu.async_copy(ctx_hbm.at[idxc_v.at[c]], rows_v, sem).wait()
            pltpu.sync_copy(rows_v, xs_hbm.at[pl.ds(base + c * CR, CR)])
        pltpu.sync_copy(p2_hbm.at[wid], idx_v)
        pltpu.async_copy(psr_hbm.at[idx_v], emb_v, sem).wait()
        pltpu.sync_copy(emb_v, ps_hbm.at[pl.ds(base, RPW)])
        pltpu.async_copy(atk_hbm.at[idx_v], emb_v, sem).wait()
        pltpu.sync_copy(emb_v, as_hbm.at[pl.ds(base, RPW)])

    return _sc_gather


@functools.lru_cache(maxsize=None)
def _make_sc_scatter():
    mesh = plsc.VectorSubcoreMesh(core_axis_name="c", subcore_axis_name="s")

    @functools.partial(
        pl.kernel,
        out_type=jax.ShapeDtypeStruct((TOK, 2 * D), jnp.float32),
        mesh=mesh,
        scratch_types=[
            pltpu.VMEM((RPW,), jnp.int32),
            pltpu.VMEM((RPW, 2 * D), jnp.float32),
            pltpu.SemaphoreType.DMA,
        ],
    )
    def _sc_scatter(blend_hbm, p2_hbm, out_hbm, idx_v, rows_v, sem):
        wid = lax.axis_index("s") * NC + lax.axis_index("c")
        base = wid * RPW
        pltpu.sync_copy(p2_hbm.at[wid], idx_v)
        pltpu.sync_copy(blend_hbm.at[pl.ds(base, RPW)], rows_v)
        pltpu.async_copy(rows_v, out_hbm.at[idx_v], sem).wait()

    return _sc_scatter


def _expert_body(x_ref, opsr_ref, oatk_ref, dW_ref, db_ref, plut_ref,
                 alut_ref, w1_ref, b1_ref, w2_ref, b2_ref, out_ref,
                 acc_ref, ent_ref):
    e = pl.program_id(0)

    @pl.when(e < E)
    def _compute():
        x = x_ref[...]
        xb = x.astype(jnp.bfloat16)
        logits = jnp.dot(xb, dW_ref[0].astype(jnp.bfloat16),
                         preferred_element_type=jnp.float32) + db_ref[0]
        m = jnp.max(logits, axis=-1, keepdims=True)
        z = logits - m
        ez = jnp.exp(z)
        s = jnp.sum(ez, axis=-1, keepdims=True)
        spt = ez / s
        pspt = z - jnp.log(s)
        ent_blk = jnp.sum(-pspt * spt) * (1.0 / (TPE * M))

        @pl.when(e == 0)
        def _():
            ent_ref[0] = 0.0

        ent_ref[0] += ent_blk

        sptb = spt.astype(jnp.bfloat16)
        psr = jnp.dot(sptb, plut_ref[0].astype(jnp.bfloat16),
                      preferred_element_type=jnp.float32)
        atk = jnp.dot(sptb, alut_ref[0].astype(jnp.bfloat16),
                      preferred_element_type=jnp.float32)

        h = jnp.maximum(jnp.dot(xb, w1_ref[...].astype(jnp.bfloat16),
                                preferred_element_type=jnp.float32)
                        + b1_ref[...], 0.0)
        u = jnp.dot(h.astype(jnp.bfloat16), w2_ref[...].astype(jnp.bfloat16),
                    preferred_element_type=jnp.float32) + b2_ref[...]
        um = jnp.max(u, axis=-1, keepdims=True)
        ue = jnp.exp(u - um)
        c = ue / jnp.sum(ue, axis=-1, keepdims=True)
        c0 = c[:, 0:1]
        c1 = c[:, 1:2]
        acc_ref[pl.ds(e * TPE, TPE), :D] = opsr_ref[...] * c0 + psr * c1
        acc_ref[pl.ds(e * TPE, TPE), D:] = oatk_ref[...] * c0 + atk * c1

    @pl.when(e == E)
    def _flush():
        out_ref[...] = acc_ref[...] + ent_ref[0]


def _tc_experts(xs, ops, oat, dec_W, dec_b, psr_lut, atk_lut, w1, b1, w2, b2):
    idx0 = lambda e: (0, 0)
    idxe = lambda e: (jnp.minimum(e, E - 1), 0)
    idxe3 = lambda e: (jnp.minimum(e, E - 1), 0, 0)
    return pl.pallas_call(
        _expert_body,
        grid=(E + 1,),
        in_specs=[
            pl.BlockSpec((TPE, HS), idxe),
            pl.BlockSpec((TPE, D), idxe),
            pl.BlockSpec((TPE, D), idxe),
            pl.BlockSpec((1, HS, M), idxe3),
            pl.BlockSpec((1, 1, M), idxe3),
            pl.BlockSpec((1, M, D), idxe3),
            pl.BlockSpec((1, M, D), idxe3),
            pl.BlockSpec((HS, 64), idx0),
            pl.BlockSpec((1, 64), idx0),
            pl.BlockSpec((64, 2), idx0),
            pl.BlockSpec((1, 2), idx0),
        ],
        out_specs=pl.BlockSpec((TOK, 2 * D), idx0),
        out_shape=jax.ShapeDtypeStruct((TOK, 2 * D), jnp.float32),
        scratch_shapes=[
            pltpu.VMEM((TOK, 2 * D), jnp.float32),
            pltpu.SMEM((1,), jnp.float32),
        ],
    )(xs, ops, oat, dec_W, dec_b, psr_lut, atk_lut, w1, b1, w2, b2)


def kernel(ctx, expert_idx, dec_W, dec_b, copy_W1, copy_b1, copy_W2, copy_b2,
           psr_lut, atk_lut, ori_psr, ori_atk):
    p = expert_idx.reshape(TOK).astype(jnp.int32)
    p3 = p.reshape(NW, CH, CR)
    p2 = p.reshape(NW, RPW)
    xs, ops, oat = _make_sc_gather()(ctx, ori_psr, ori_atk, p3, p2)
    blended = _tc_experts(
        xs, ops, oat, dec_W, dec_b.reshape(E, 1, M), psr_lut, atk_lut,
        copy_W1, copy_b1.reshape(1, 64), copy_W2, copy_b2.reshape(1, 2))
    return _make_sc_scatter()(blended, p2)


# speedup vs baseline: 3.3195x; 1.0313x over previous
import functools

import jax
import jax.numpy as jnp
from jax import lax
from jax.experimental import pallas as pl
from jax.experimental.pallas import tpu as pltpu
from jax.experimental.pallas import tpu_sc as plsc

TOK = 4096
HS = 1024
E = 8
M = 512
D = 256
TPE = TOK // E

NC = 2
NS = 16
NW = NC * NS
RPW = TOK // NW
CH = 8
CR = RPW // CH

@functools.lru_cache(maxsize=None)
def _make_sc_gather():
    mesh = plsc.VectorSubcoreMesh(core_axis_name="c", subcore_axis_name="s")

    @functools.partial(
        pl.kernel,
        out_type=[
            jax.ShapeDtypeStruct((TOK, HS), jnp.float32),
            jax.ShapeDtypeStruct((TOK, D), jnp.float32),
            jax.ShapeDtypeStruct((TOK, D), jnp.float32),
        ],
        mesh=mesh,
        scratch_types=[
            pltpu.VMEM((CH, CR), jnp.int32),
            pltpu.VMEM((RPW,), jnp.int32),
            pltpu.VMEM((CR, HS), jnp.float32),
            pltpu.VMEM((CR, HS), jnp.float32),
            pltpu.VMEM((RPW, D), jnp.float32),
            pltpu.VMEM((RPW, D), jnp.float32),
            pltpu.SemaphoreType.DMA,
            pltpu.SemaphoreType.DMA,
            pltpu.SemaphoreType.DMA,
            pltpu.SemaphoreType.DMA,
        ],
    )
    def _sc_gather(ctx_hbm, psr_hbm, atk_hbm, p3_hbm, p2_hbm,
                   xs_hbm, ps_hbm, as_hbm,
                   idxc_v, idx_v, buf_a, buf_b, emb_p, emb_a,
                   sem_a, sem_b, sem_p, sem_k):
        wid = lax.axis_index("s") * NC + lax.axis_index("c")
        base = wid * RPW
        pltpu.sync_copy(p3_hbm.at[wid], idxc_v)
        pltpu.sync_copy(p2_hbm.at[wid], idx_v)
        cp_p = pltpu.async_copy(psr_hbm.at[idx_v], emb_p, sem_p)
        cp_k = pltpu.async_copy(atk_hbm.at[idx_v], emb_a, sem_k)
        bufs = (buf_a, buf_b)
        sems = (sem_a, sem_b)
        gathers = [pltpu.async_copy(ctx_hbm.at[idxc_v.at[0]], buf_a, sem_a),
                   pltpu.async_copy(ctx_hbm.at[idxc_v.at[1]], buf_b, sem_b)]
        for c in range(CH):
            gathers[c % 2].wait()
            pltpu.sync_copy(bufs[c % 2], xs_hbm.at[pl.ds(base + c * CR, CR)])
            if c + 2 < CH:
                gathers[c % 2] = pltpu.async_copy(
                    ctx_hbm.at[idxc_v.at[c + 2]], bufs[c % 2], sems[c % 2])
        cp_p.wait()
        pltpu.sync_copy(emb_p, ps_hbm.at[pl.ds(base, RPW)])
        cp_k.wait()
        pltpu.sync_copy(emb_a, as_hbm.at[pl.ds(base, RPW)])

    return _sc_gather


@functools.lru_cache(maxsize=None)
def _make_sc_scatter():
    mesh = plsc.VectorSubcoreMesh(core_axis_name="c", subcore_axis_name="s")
    HR = RPW // 2

    @functools.partial(
        pl.kernel,
        out_type=jax.ShapeDtypeStruct((TOK, 2 * D), jnp.float32),
        mesh=mesh,
        scratch_types=[
            pltpu.VMEM((2, HR), jnp.int32),
            pltpu.VMEM((HR, 2 * D), jnp.float32),
            pltpu.VMEM((HR, 2 * D), jnp.float32),
            pltpu.SemaphoreType.DMA,
            pltpu.SemaphoreType.DMA,
        ],
    )
    def _sc_scatter(blend_hbm, p2s_hbm, out_hbm, idx_v, buf_a, buf_b,
                    sem_a, sem_b):
        wid = lax.axis_index("s") * NC + lax.axis_index("c")
        base = wid * RPW
        pltpu.sync_copy(p2s_hbm.at[wid], idx_v)
        pltpu.sync_copy(blend_hbm.at[pl.ds(base, HR)], buf_a)
        cp_a = pltpu.async_copy(buf_a, out_hbm.at[idx_v.at[0]], sem_a)
        pltpu.sync_copy(blend_hbm.at[pl.ds(base + HR, HR)], buf_b)
        cp_b = pltpu.async_copy(buf_b, out_hbm.at[idx_v.at[1]], sem_b)
        cp_a.wait()
        cp_b.wait()

    return _sc_scatter


def _expert_body(x_ref, opsr_ref, oatk_ref, dW_ref, db_ref, plut_ref,
                 alut_ref, w1_ref, b1_ref, w2_ref, b2_ref, out_ref,
                 acc_ref, ent_ref):
    e = pl.program_id(0)

    @pl.when(e < E)
    def _compute():
        x = x_ref[...]
        xb = x.astype(jnp.bfloat16)
        logits = jnp.dot(xb, dW_ref[0].astype(jnp.bfloat16),
                         preferred_element_type=jnp.float32) + db_ref[0]
        m = jnp.max(logits, axis=-1, keepdims=True)
        z = logits - m
        ez = jnp.exp(z)
        s = jnp.sum(ez, axis=-1, keepdims=True)
        spt = ez / s
        pspt = z - jnp.log(s)
        ent_blk = jnp.sum(-pspt * spt) * (1.0 / (TPE * M))

        @pl.when(e == 0)
        def _():
            ent_ref[0] = 0.0

        ent_ref[0] += ent_blk

        sptb = spt.astype(jnp.bfloat16)
        psr = jnp.dot(sptb, plut_ref[0].astype(jnp.bfloat16),
                      preferred_element_type=jnp.float32)
        atk = jnp.dot(sptb, alut_ref[0].astype(jnp.bfloat16),
                      preferred_element_type=jnp.float32)

        h = jnp.maximum(jnp.dot(xb, w1_ref[...].astype(jnp.bfloat16),
                                preferred_element_type=jnp.float32)
                        + b1_ref[...], 0.0)
        u = jnp.dot(h.astype(jnp.bfloat16), w2_ref[...].astype(jnp.bfloat16),
                    preferred_element_type=jnp.float32) + b2_ref[...]
        um = jnp.max(u, axis=-1, keepdims=True)
        ue = jnp.exp(u - um)
        c = ue / jnp.sum(ue, axis=-1, keepdims=True)
        c0 = c[:, 0:1]
        c1 = c[:, 1:2]
        acc_ref[pl.ds(e * TPE, TPE), :D] = opsr_ref[...] * c0 + psr * c1
        acc_ref[pl.ds(e * TPE, TPE), D:] = oatk_ref[...] * c0 + atk * c1

    @pl.when(e == E)
    def _flush():
        out_ref[...] = acc_ref[...] + ent_ref[0]


def _tc_experts(xs, ops, oat, dec_W, dec_b, psr_lut, atk_lut, w1, b1, w2, b2):
    idx0 = lambda e: (0, 0)
    idxe = lambda e: (jnp.minimum(e, E - 1), 0)
    idxe3 = lambda e: (jnp.minimum(e, E - 1), 0, 0)
    return pl.pallas_call(
        _expert_body,
        grid=(E + 1,),
        in_specs=[
            pl.BlockSpec((TPE, HS), idxe),
            pl.BlockSpec((TPE, D), idxe),
            pl.BlockSpec((TPE, D), idxe),
            pl.BlockSpec((1, HS, M), idxe3),
            pl.BlockSpec((1, 1, M), idxe3),
            pl.BlockSpec((1, M, D), idxe3),
            pl.BlockSpec((1, M, D), idxe3),
            pl.BlockSpec((HS, 64), idx0),
            pl.BlockSpec((1, 64), idx0),
            pl.BlockSpec((64, 2), idx0),
            pl.BlockSpec((1, 2), idx0),
        ],
        out_specs=pl.BlockSpec((TOK, 2 * D), idx0),
        out_shape=jax.ShapeDtypeStruct((TOK, 2 * D), jnp.float32),
        scratch_shapes=[
            pltpu.VMEM((TOK, 2 * D), jnp.float32),
            pltpu.SMEM((1,), jnp.float32),
        ],
    )(xs, ops, oat, dec_W, dec_b, psr_lut, atk_lut, w1, b1, w2, b2)


def kernel(ctx, expert_idx, dec_W, dec_b, copy_W1, copy_b1, copy_W2, copy_b2,
           psr_lut, atk_lut, ori_psr, ori_atk):
    p = expert_idx.reshape(TOK).astype(jnp.int32)
    p3 = p.reshape(NW, CH, CR)
    p2 = p.reshape(NW, RPW)
    p2s = p.reshape(NW, 2, RPW // 2)
    xs, ops, oat = _make_sc_gather()(ctx, ori_psr, ori_atk, p3, p2)
    blended = _tc_experts(
        xs, ops, oat, dec_W, dec_b.reshape(E, 1, M), psr_lut, atk_lut,
        copy_W1, copy_b1.reshape(1, 64), copy_W2, copy_b2.reshape(1, 2))
    return _make_sc_scatter()(blended, p2s)


# speedup vs baseline: 3.3798x; 1.0181x over previous
import functools

import jax
import jax.numpy as jnp
from jax import lax
from jax.experimental import pallas as pl
from jax.experimental.pallas import tpu as pltpu
from jax.experimental.pallas import tpu_sc as plsc

TOK = 4096
HS = 1024
E = 8
M = 512
D = 256
TPE = TOK // E
EH = E // 2
HTOK = TOK // 2

NC = 2
NS = 16
NW = NC * NS
RPW = HTOK // NW
WPE = TPE // RPW


@functools.lru_cache(maxsize=None)
def _make_sc_gather(h):
    mesh = plsc.VectorSubcoreMesh(core_axis_name="c", subcore_axis_name="s")

    @functools.partial(
        pl.kernel,
        out_type=[
            jax.ShapeDtypeStruct((HTOK, HS), jnp.float32),
            jax.ShapeDtypeStruct((HTOK, D), jnp.float32),
            jax.ShapeDtypeStruct((HTOK, D), jnp.float32),
        ],
        mesh=mesh,
        scratch_types=[
            pltpu.VMEM((RPW,), jnp.int32),
            pltpu.VMEM((RPW // 2, HS), jnp.float32),
            pltpu.VMEM((RPW // 2, HS), jnp.float32),
            pltpu.VMEM((RPW, D), jnp.float32),
            pltpu.VMEM((RPW, D), jnp.float32),
            pltpu.SemaphoreType.DMA,
            pltpu.SemaphoreType.DMA,
            pltpu.SemaphoreType.DMA,
            pltpu.SemaphoreType.DMA,
        ],
    )
    def _sc_gather(ctx_hbm, psr_hbm, atk_hbm, eidx_hbm,
                   xs_hbm, ps_hbm, as_hbm,
                   idx_v, buf_a, buf_b, emb_p, emb_a,
                   sem_a, sem_b, sem_p, sem_k):
        wid = lax.axis_index("s") * NC + lax.axis_index("c")
        e = h * EH + wid // WPE
        col = (wid % WPE) * RPW
        base = wid * RPW
        half = RPW // 2
        pltpu.sync_copy(eidx_hbm.at[e, pl.ds(col, RPW)], idx_v)
        cp_p = pltpu.async_copy(psr_hbm.at[idx_v], emb_p, sem_p)
        cp_k = pltpu.async_copy(atk_hbm.at[idx_v], emb_a, sem_k)
        cp_a = pltpu.async_copy(ctx_hbm.at[idx_v.at[pl.ds(0, half)]],
                                buf_a, sem_a)
        cp_b = pltpu.async_copy(ctx_hbm.at[idx_v.at[pl.ds(half, half)]],
                                buf_b, sem_b)
        cp_a.wait()
        pltpu.sync_copy(buf_a, xs_hbm.at[pl.ds(base, half)])
        cp_b.wait()
        pltpu.sync_copy(buf_b, xs_hbm.at[pl.ds(base + half, half)])
        cp_p.wait()
        pltpu.sync_copy(emb_p, ps_hbm.at[pl.ds(base, RPW)])
        cp_k.wait()
        pltpu.sync_copy(emb_a, as_hbm.at[pl.ds(base, RPW)])

    return _sc_gather


@functools.lru_cache(maxsize=None)
def _make_sc_scatter():
    mesh = plsc.VectorSubcoreMesh(core_axis_name="c", subcore_axis_name="s")

    @functools.partial(
        pl.kernel,
        out_type=jax.ShapeDtypeStruct((TOK, 2 * D), jnp.float32),
        mesh=mesh,
        scratch_types=[
            pltpu.VMEM((RPW,), jnp.int32),
            pltpu.VMEM((RPW,), jnp.int32),
            pltpu.VMEM((RPW, 2 * D), jnp.float32),
            pltpu.VMEM((RPW, 2 * D), jnp.float32),
            pltpu.VMEM((1, 16), jnp.float32),
            pltpu.VMEM((1, 16), jnp.float32),
            pltpu.SemaphoreType.DMA,
            pltpu.SemaphoreType.DMA,
            pltpu.SemaphoreType.DMA,
            pltpu.SemaphoreType.DMA,
        ],
    )
    def _sc_scatter(b1_hbm, b2_hbm, eidx_hbm, ent1_hbm, ent2_hbm, out_hbm,
                    idx_a, idx_b, buf_a, buf_b, ent1_v, ent2_v,
                    sem_a, sem_b, sem_la, sem_lb):
        wid = lax.axis_index("s") * NC + lax.axis_index("c")
        eh = wid // WPE
        col = (wid % WPE) * RPW
        base = wid * RPW
        pltpu.sync_copy(ent1_hbm, ent1_v)
        pltpu.sync_copy(ent2_hbm, ent2_v)
        ent = ent1_v[0] + ent2_v[0]
        pltpu.sync_copy(eidx_hbm.at[eh, pl.ds(col, RPW)], idx_a)
        pltpu.sync_copy(eidx_hbm.at[EH + eh, pl.ds(col, RPW)], idx_b)
        cp_la = pltpu.async_copy(b1_hbm.at[pl.ds(base, RPW)], buf_a, sem_la)
        cp_lb = pltpu.async_copy(b2_hbm.at[pl.ds(base, RPW)], buf_b, sem_lb)

        def _add_ent(buf):
            def _row(r, _):
                for j in range(2 * D // 16):
                    buf[r, pl.ds(j * 16, 16)] += ent
                return 0
            lax.fori_loop(0, RPW, _row, 0)

        cp_la.wait()
        _add_ent(buf_a)
        cp_a = pltpu.async_copy(buf_a, out_hbm.at[idx_a], sem_a)
        cp_lb.wait()
        _add_ent(buf_b)
        cp_b = pltpu.async_copy(buf_b, out_hbm.at[idx_b], sem_b)
        cp_a.wait()
        cp_b.wait()

    return _sc_scatter


def _make_expert_body(h):
    def _expert_body(x_ref, opsr_ref, oatk_ref, dW_ref, db_ref, plut_ref,
                     alut_ref, w1_ref, b1_ref, w2_ref, b2_ref,
                     out_ref, ent_ref, acc_ref):
        e = pl.program_id(0)
        xb = x_ref[...].astype(jnp.bfloat16)
        logits = jnp.dot(xb, dW_ref[0].astype(jnp.bfloat16),
                         preferred_element_type=jnp.float32) + db_ref[0]
        m = jnp.max(logits, axis=-1, keepdims=True)
        z = logits - m
        ez = jnp.exp(z)
        s = jnp.sum(ez, axis=-1, keepdims=True)
        spt = ez / s
        pspt = z - jnp.log(s)
        ent_blk = jnp.sum(-pspt * spt) * (1.0 / (TPE * M))

        @pl.when(e == 0)
        def _():
            acc_ref[0] = 0.0

        acc_ref[0] += ent_blk

        sptb = spt.astype(jnp.bfloat16)
        psr = jnp.dot(sptb, plut_ref[0].astype(jnp.bfloat16),
                      preferred_element_type=jnp.float32)
        atk = jnp.dot(sptb, alut_ref[0].astype(jnp.bfloat16),
                      preferred_element_type=jnp.float32)

        hh = jnp.maximum(jnp.dot(xb, w1_ref[...].astype(jnp.bfloat16),
                                 preferred_element_type=jnp.float32)
                         + b1_ref[...], 0.0)
        u = jnp.dot(hh.astype(jnp.bfloat16), w2_ref[...].astype(jnp.bfloat16),
                    preferred_element_type=jnp.float32) + b2_ref[...]
        um = jnp.max(u, axis=-1, keepdims=True)
        ue = jnp.exp(u - um)
        c = ue / jnp.sum(ue, axis=-1, keepdims=True)
        c0 = c[:, 0:1]
        c1 = c[:, 1:2]
        out_ref[:, :D] = opsr_ref[...] * c0 + psr * c1
        out_ref[:, D:] = oatk_ref[...] * c0 + atk * c1

        @pl.when(e == EH - 1)
        def _():
            ent_ref[...] = jnp.full((1, 16), acc_ref[0], jnp.float32)

    return _expert_body


@functools.lru_cache(maxsize=None)
def _make_tc_experts(h):
    idx0 = lambda e: (0, 0)
    idxe = lambda e: (e, 0)
    idxe3 = lambda e: (h * EH + e, 0, 0)
    body = _make_expert_body(h)

    def _call(xs, ops, oat, dec_W, dec_b, psr_lut, atk_lut, w1, b1, w2, b2):
        return pl.pallas_call(
            body,
            grid=(EH,),
            in_specs=[
                pl.BlockSpec((TPE, HS), idxe),
                pl.BlockSpec((TPE, D), idxe),
                pl.BlockSpec((TPE, D), idxe),
                pl.BlockSpec((1, HS, M), idxe3),
                pl.BlockSpec((1, 1, M), idxe3),
                pl.BlockSpec((1, M, D), idxe3),
                pl.BlockSpec((1, M, D), idxe3),
                pl.BlockSpec((HS, 64), idx0),
                pl.BlockSpec((1, 64), idx0),
                pl.BlockSpec((64, 2), idx0),
                pl.BlockSpec((1, 2), idx0),
            ],
            out_specs=[
                pl.BlockSpec((TPE, 2 * D), idxe),
                pl.BlockSpec((1, 16), idx0),
            ],
            out_shape=[
                jax.ShapeDtypeStruct((HTOK, 2 * D), jnp.float32),
                jax.ShapeDtypeStruct((1, 16), jnp.float32),
            ],
            scratch_shapes=[pltpu.SMEM((1,), jnp.float32)],
        )(xs, ops, oat, dec_W, dec_b, psr_lut, atk_lut, w1, b1, w2, b2)

    return _call


def kernel(ctx, expert_idx, dec_W, dec_b, copy_W1, copy_b1, copy_W2, copy_b2,
           psr_lut, atk_lut, ori_psr, ori_atk):
    eidx = expert_idx.astype(jnp.int32)
    db3 = dec_b.reshape(E, 1, M)
    b1r = copy_b1.reshape(1, 64)
    b2r = copy_b2.reshape(1, 2)
    xs1, ops1, oat1 = _make_sc_gather(0)(ctx, ori_psr, ori_atk, eidx)
    xs2, ops2, oat2 = _make_sc_gather(1)(ctx, ori_psr, ori_atk, eidx)
    b1, ent1 = _make_tc_experts(0)(xs1, ops1, oat1, dec_W, db3, psr_lut,
                                   atk_lut, copy_W1, b1r, copy_W2, b2r)
    b2, ent2 = _make_tc_experts(1)(xs2, ops2, oat2, dec_W, db3, psr_lut,
                                   atk_lut, copy_W1, b1r, copy_W2, b2r)
    return _make_sc_scatter()(b1, b2, eidx, ent1, ent2)
